# baseline (speedup 1.0000x reference)
import functools

import jax
import jax.numpy as jnp
from jax import lax
from jax.experimental import pallas as pl
from jax.experimental.pallas import tpu as pltpu
from jax.experimental.pallas import tpu_sc as plsc

E = 320000
H = 128
NC = 2
NS = 16
NW = NC * NS
EPW = E // NW
K = 80
NCHUNK = EPW // K
G = K // 16

_mesh = plsc.VectorSubcoreMesh(core_axis_name="c", subcore_axis_name="s")


@functools.partial(
    pl.kernel,
    mesh=_mesh,
    out_type=jax.ShapeDtypeStruct((E,), jnp.float32),
    compiler_params=pltpu.CompilerParams(needs_layout_passes=False),
    scratch_types=[
        pltpu.VMEM((EPW,), jnp.int32),
        pltpu.VMEM((EPW,), jnp.int32),
        pltpu.VMEM((EPW,), jnp.int32),
        pltpu.VMEM((EPW,), jnp.float32),
        pltpu.VMEM((K, H), jnp.float32),
        pltpu.VMEM((K, H), jnp.float32),
        pltpu.VMEM((K, H), jnp.float32),
        pltpu.VMEM((K, H), jnp.float32),
        pltpu.VMEM((K, H), jnp.float32),
        pltpu.VMEM((K, H), jnp.float32),
        pltpu.SemaphoreType.DMA,
        pltpu.SemaphoreType.DMA,
        pltpu.SemaphoreType.DMA,
        pltpu.SemaphoreType.DMA,
        pltpu.SemaphoreType.DMA,
        pltpu.SemaphoreType.DMA,
    ],
)
def _distmult_sc(src_hbm, dst_hbm, typ_hbm, z_hbm, rel_hbm, out_hbm,
                 sidx_v, didx_v, tidx_v, out_v,
                 zsA, zdA, rlA, zsB, zdB, rlB,
                 ssA, sdA, srA, ssB, sdB, srB):
    wid = lax.axis_index("s") * NC + lax.axis_index("c")
    row16 = lax.iota(jnp.int32, 16)
    bufs = ((zsA, zdA, rlA, ssA, sdA, srA),
            (zsB, zdB, rlB, ssB, sdB, srB))

    base = wid * EPW
    pltpu.sync_copy(src_hbm.at[pl.ds(base, EPW)], sidx_v)
    pltpu.sync_copy(dst_hbm.at[pl.ds(base, EPW)], didx_v)
    pltpu.sync_copy(typ_hbm.at[pl.ds(base, EPW)], tidx_v)

    def start(c, buf):
        zs, zd, rl, s_s, s_d, s_r = buf
        off = c * K
        pltpu.async_copy(z_hbm.at[sidx_v.at[pl.ds(off, K)]], zs, s_s)
        pltpu.async_copy(z_hbm.at[didx_v.at[pl.ds(off, K)]], zd, s_d)
        pltpu.async_copy(rel_hbm.at[tidx_v.at[pl.ds(off, K)]], rl, s_r)

    def wait(c, buf):
        zs, zd, rl, s_s, s_d, s_r = buf
        off = c * K
        pltpu.make_async_copy(z_hbm.at[sidx_v.at[pl.ds(off, K)]], zs, s_s).wait()
        pltpu.make_async_copy(z_hbm.at[didx_v.at[pl.ds(off, K)]], zd, s_d).wait()
        pltpu.make_async_copy(rel_hbm.at[tidx_v.at[pl.ds(off, K)]], rl, s_r).wait()

    start(0, bufs[0])

    def chunk_pair(i, carry):
        for par in range(2):
            c = 2 * i + par
            nxt = c + 1
            start(nxt, bufs[(par + 1) % 2])
            wait(c, bufs[par])
            _compute_chunk(c, bufs[par], out_v, row16)
        return carry

    lax.fori_loop(0, (NCHUNK - 1) // 2, chunk_pair, 0)
    cl = NCHUNK - 1
    wait(cl, bufs[0])
    _compute_chunk(cl, bufs[0], out_v, row16)

    pltpu.sync_copy(out_v, out_hbm.at[pl.ds(base, EPW)])


_DU = 4


def _compute_chunk(c, buf, out_v, row16):
    zs, zd, rl = buf[0], buf[1], buf[2]

    def group_body(g, carry):
        ridx = row16 + g * 16

        def dim_body(t, accs):
            cbase = lax.broadcast(t * _DU, (16,))
            new = []
            for k in range(_DU):
                cidx = cbase + k
                a = plsc.load_gather(zs, [ridx, cidx])
                b = plsc.load_gather(rl, [ridx, cidx])
                d = plsc.load_gather(zd, [ridx, cidx])
                new.append(accs[k] + a * b * d)
            return tuple(new)

        zero = jnp.zeros((16,), jnp.float32)
        accs = lax.fori_loop(0, H // _DU, dim_body, (zero,) * _DU)
        acc = accs[0]
        for k in range(1, _DU):
            acc = acc + accs[k]
        out_v[pl.ds(c * K + g * 16, 16)] = acc
        return carry

    lax.fori_loop(0, G, group_body, 0)


def kernel(z, edge_index, edge_type, rel_emb):
    src = edge_index[0].astype(jnp.int32)
    dst = edge_index[1].astype(jnp.int32)
    typ = edge_type.astype(jnp.int32)
    return _distmult_sc(src, dst, typ, z, rel_emb)


# speedup vs baseline: 7.2455x; 7.2455x over previous
import functools

import jax
import jax.numpy as jnp
from jax import lax
from jax.experimental import pallas as pl
from jax.experimental.pallas import tpu as pltpu
from jax.experimental.pallas import tpu_sc as plsc

E = 320000
H = 128
NC = 2
NS = 16
NW = NC * NS
EPW = E // NW
K = 80
NCHUNK = EPW // K
G = K // 16

_mesh = plsc.VectorSubcoreMesh(core_axis_name="c", subcore_axis_name="s")


@functools.partial(
    pl.kernel,
    mesh=_mesh,
    out_type=jax.ShapeDtypeStruct((E,), jnp.float32),
    compiler_params=pltpu.CompilerParams(needs_layout_passes=False),
    scratch_types=[
        pltpu.VMEM((EPW,), jnp.int32),
        pltpu.VMEM((EPW,), jnp.int32),
        pltpu.VMEM((EPW,), jnp.int32),
        pltpu.VMEM((EPW,), jnp.float32),
        pltpu.VMEM((K, H), jnp.float32),
        pltpu.VMEM((K, H), jnp.float32),
        pltpu.VMEM((K, H), jnp.float32),
        pltpu.VMEM((K, H), jnp.float32),
        pltpu.VMEM((K, H), jnp.float32),
        pltpu.VMEM((K, H), jnp.float32),
        pltpu.SemaphoreType.DMA,
        pltpu.SemaphoreType.DMA,
        pltpu.SemaphoreType.DMA,
        pltpu.SemaphoreType.DMA,
        pltpu.SemaphoreType.DMA,
        pltpu.SemaphoreType.DMA,
    ],
)
def _distmult_sc(src_hbm, dst_hbm, typ_hbm, z_hbm, rel_hbm, out_hbm,
                 sidx_v, didx_v, tidx_v, out_v,
                 zsA, zdA, rlA, zsB, zdB, rlB,
                 ssA, sdA, srA, ssB, sdB, srB):
    wid = lax.axis_index("s") * NC + lax.axis_index("c")
    row16 = lax.iota(jnp.int32, 16)
    bufs = ((zsA, zdA, rlA, ssA, sdA, srA),
            (zsB, zdB, rlB, ssB, sdB, srB))

    base = wid * EPW
    pltpu.sync_copy(src_hbm.at[pl.ds(base, EPW)], sidx_v)
    pltpu.sync_copy(dst_hbm.at[pl.ds(base, EPW)], didx_v)
    pltpu.sync_copy(typ_hbm.at[pl.ds(base, EPW)], tidx_v)

    def start(c, buf):
        zs, zd, rl, s_s, s_d, s_r = buf
        off = c * K
        pltpu.async_copy(z_hbm.at[sidx_v.at[pl.ds(off, K)]], zs, s_s)
        pltpu.async_copy(z_hbm.at[didx_v.at[pl.ds(off, K)]], zd, s_d)
        pltpu.async_copy(rel_hbm.at[tidx_v.at[pl.ds(off, K)]], rl, s_r)

    def wait(c, buf):
        zs, zd, rl, s_s, s_d, s_r = buf
        off = c * K
        pltpu.make_async_copy(z_hbm.at[sidx_v.at[pl.ds(off, K)]], zs, s_s).wait()
        pltpu.make_async_copy(z_hbm.at[didx_v.at[pl.ds(off, K)]], zd, s_d).wait()
        pltpu.make_async_copy(rel_hbm.at[tidx_v.at[pl.ds(off, K)]], rl, s_r).wait()

    start(0, bufs[0])

    def chunk_pair(i, carry):
        for par in range(2):
            c = 2 * i + par
            nxt = c + 1
            start(nxt, bufs[(par + 1) % 2])
            wait(c, bufs[par])
            _compute_chunk(c, bufs[par], out_v, row16)
        return carry

    lax.fori_loop(0, (NCHUNK - 1) // 2, chunk_pair, 0)
    cl = NCHUNK - 1
    wait(cl, bufs[0])
    _compute_chunk(cl, bufs[0], out_v, row16)

    pltpu.sync_copy(out_v, out_hbm.at[pl.ds(base, EPW)])


_EU = 2


def _compute_chunk(c, buf, out_v, row16):
    zs, zd, rl = buf[0], buf[1], buf[2]

    def group_body(g, carry):
        gbase = g * 16

        def edge_blk(eb, acc_out):
            for u in range(_EU):
                e16 = eb * _EU + u
                e = gbase + e16
                prods = []
                for j in range(H // 16):
                    sl = pl.ds(j * 16, 16)
                    prods.append(zs[e, sl] * rl[e, sl] * zd[e, sl])
                while len(prods) > 1:
                    prods = [prods[k] + prods[k + 1]
                             for k in range(0, len(prods), 2)]
                s = jnp.sum(prods[0])
                acc_out = jnp.where(row16 == e16, s, acc_out)
            return acc_out

        acc_out = lax.fori_loop(0, 16 // _EU, edge_blk,
                                jnp.zeros((16,), jnp.float32))
        out_v[pl.ds(c * K + gbase, 16)] = acc_out
        return carry

    lax.fori_loop(0, G, group_body, 0)


def kernel(z, edge_index, edge_type, rel_emb):
    src = edge_index[0].astype(jnp.int32)
    dst = edge_index[1].astype(jnp.int32)
    typ = edge_type.astype(jnp.int32)
    return _distmult_sc(src, dst, typ, z, rel_emb)
